# Initial kernel scaffold; baseline (speedup 1.0000x reference)
import jax
import jax.numpy as jnp
from jax.experimental import pallas as pl


def kernel(x, edge_index, edge_attr, batch_index, solvent_descriptors, mol_fingerprints, W1, b1, W2, b2, lin1_W, lin1_b, lin2_W, lin2_b, lin3_W, lin3_b, heads_W1, heads_b1, heads_W2, heads_b2):
    raise NotImplementedError("write your pallas kernel here")


# speedup vs baseline: 16.1429x; 16.1429x over previous
import functools

import jax
import jax.numpy as jnp
from jax import lax
from jax.experimental import pallas as pl
from jax.experimental.pallas import tpu as pltpu
from jax.experimental.pallas import tpu_sc as plsc

NC = 2
NS = 16
LANES = 16
CHUNK = 128


def _sc_mesh():
    return plsc.VectorSubcoreMesh(core_axis_name="c", subcore_axis_name="s")


def _zero_fill(ref, rows, width):
    def body(i, carry):
        for k in range(width // LANES):
            ref[i, pl.ds(k * LANES, LANES)] = jnp.zeros((LANES,), jnp.float32)
        return carry
    lax.fori_loop(0, rows, body, 0)


def _zero_shared_rows(fill_v, acc_s, base, rpt):
    nfull, rem = rpt // 128, rpt % 128
    for k in range(nfull):
        pltpu.sync_copy(fill_v, acc_s.at[pl.ds(base + 128 * k, 128)])
    if rem:
        pltpu.sync_copy(fill_v.at[pl.ds(0, rem)],
                        acc_s.at[pl.ds(base + 128 * nfull, rem)])


def _make_deg_kernel(np_rows, ch_per_tile, rpt, width=16):
    @functools.partial(
        pl.kernel,
        out_type=jax.ShapeDtypeStruct((NC, np_rows, width), jnp.float32),
        mesh=_sc_mesh(),
        scratch_types=[
            pltpu.VMEM((ch_per_tile, CHUNK), jnp.int32),
            pltpu.VMEM((128, width), jnp.float32),
            pltpu.VMEM((128, width), jnp.float32),
            pltpu.VMEM_SHARED((np_rows, width), jnp.float32),
        ],
    )
    def deg_kernel(dst_hbm, ones_hbm, out_hbm, idx_v, fill_v, ones_v, acc_s):
        c = lax.axis_index("c")
        s = lax.axis_index("s")
        wid = s * NC + c
        base = s * rpt
        _zero_fill(fill_v, 128, width)
        _zero_shared_rows(fill_v, acc_s, base, rpt)
        pltpu.sync_copy(ones_hbm, ones_v)
        plsc.subcore_barrier()
        pltpu.sync_copy(dst_hbm.at[wid], idx_v)

        def step(j, carry):
            pltpu.sync_copy(ones_v, acc_s.at[idx_v.at[j]], add=True)
            return carry
        lax.fori_loop(0, ch_per_tile, step, 0)
        plsc.subcore_barrier()
        pltpu.sync_copy(acc_s.at[pl.ds(base, rpt)],
                        out_hbm.at[c].at[pl.ds(base, rpt)])

    return deg_kernel


def _make_agg_kernel(np_rows, ch_per_tile, rpt, width):
    @functools.partial(
        pl.kernel,
        out_type=jax.ShapeDtypeStruct((NC, np_rows, width), jnp.float32),
        mesh=_sc_mesh(),
        scratch_types=[
            pltpu.VMEM((ch_per_tile, CHUNK), jnp.int32),
            pltpu.VMEM((ch_per_tile, CHUNK), jnp.int32),
            pltpu.VMEM((128, width), jnp.float32),
            pltpu.VMEM_SHARED((np_rows, width), jnp.float32),
            pltpu.SemaphoreType.DMA,
        ],
    )
    def agg_kernel(src_hbm, dst_hbm, tab_hbm, out_hbm,
                   idxs_v, idxd_v, rows_v, acc_s, sem):
        c = lax.axis_index("c")
        s = lax.axis_index("s")
        wid = s * NC + c
        base = s * rpt
        _zero_fill(rows_v, 128, width)
        _zero_shared_rows(rows_v, acc_s, base, rpt)
        plsc.subcore_barrier()
        pltpu.sync_copy(src_hbm.at[wid], idxs_v)
        pltpu.sync_copy(dst_hbm.at[wid], idxd_v)

        def step(j, carry):
            pltpu.async_copy(tab_hbm.at[idxs_v.at[j]], rows_v, sem).wait()
            pltpu.sync_copy(rows_v, acc_s.at[idxd_v.at[j]], add=True)
            return carry
        lax.fori_loop(0, ch_per_tile, step, 0)
        plsc.subcore_barrier()
        pltpu.sync_copy(acc_s.at[pl.ds(base, rpt)],
                        out_hbm.at[c].at[pl.ds(base, rpt)])

    return agg_kernel


def _dinv_from_deg(degt_ref):
    deg = degt_ref[0, :, 0:1] + degt_ref[1, :, 0:1]
    return lax.rsqrt(jnp.maximum(deg, 1e-12))


def _tc_prescale(x_ref, w1_ref, degt_ref, xs_ref):
    dinv = _dinv_from_deg(degt_ref)
    xw = jnp.dot(x_ref[...], w1_ref[...], preferred_element_type=jnp.float32)
    xs_ref[...] = xw * dinv


def _tc_layer2(agg_ref, degt_ref, b1_ref, ys_ref):
    dinv = _dinv_from_deg(degt_ref)
    h1 = jnp.maximum(dinv * (agg_ref[0] + agg_ref[1]) + b1_ref[...], 0.0)
    ys_ref[...] = h1 * dinv


def _tc_head(agg2_ref, degt_ref, batch_ref, solv_ref, w2_ref, b2_ref,
             l1a_ref, l1b_ref, l1bias_ref, l2w_ref, l2bias_ref,
             l3w_ref, l3bias_ref, w1cat_ref, b1cat_ref, w2blk_ref, b2row_ref,
             out_ref):
    num_graphs = out_ref.shape[0]
    np_rows = batch_ref.shape[1]
    dinv = _dinv_from_deg(degt_ref)
    nodes = dinv * (agg2_ref[0] + agg2_ref[1])
    gids = lax.broadcasted_iota(jnp.int32, (num_graphs, np_rows), 0)
    mask = (batch_ref[...] == gids).astype(jnp.float32)
    pooled_pre = jnp.dot(mask, nodes, preferred_element_type=jnp.float32)
    cnt = jnp.sum(mask, axis=1, keepdims=True)
    pooled = jnp.dot(pooled_pre, w2_ref[...],
                     preferred_element_type=jnp.float32) + cnt * b2_ref[...]
    z = jnp.dot(pooled, l1a_ref[...], preferred_element_type=jnp.float32)
    z = z + jnp.dot(solv_ref[...], l1b_ref[...],
                    preferred_element_type=jnp.float32)
    z = jnp.maximum(z + l1bias_ref[...], 0.0)
    z = jnp.maximum(jnp.dot(z, l2w_ref[...],
                            preferred_element_type=jnp.float32)
                    + l2bias_ref[...], 0.0)
    z = jnp.maximum(jnp.dot(z, l3w_ref[...],
                            preferred_element_type=jnp.float32)
                    + l3bias_ref[...], 0.0)
    hh = jnp.maximum(jnp.dot(z, w1cat_ref[...],
                             preferred_element_type=jnp.float32)
                     + b1cat_ref[...], 0.0)
    out_ref[...] = jnp.dot(hh, w2blk_ref[...],
                           preferred_element_type=jnp.float32) + b2row_ref[...]


def kernel(x, edge_index, edge_attr, batch_index, solvent_descriptors,
           mol_fingerprints, W1, b1, W2, b2, lin1_W, lin1_b, lin2_W, lin2_b,
           lin3_W, lin3_b, heads_W1, heads_b1, heads_W2, heads_b2):
    n, din = x.shape
    e = edge_index.shape[1]
    g = solvent_descriptors.shape[0]
    d1 = W1.shape[1]
    d2 = W2.shape[1]

    rpt = -(-(n + 1) // (NS * 8)) * 8
    np_rows = rpt * NS

    loop = jnp.arange(n, dtype=edge_index.dtype)
    src = jnp.concatenate([edge_index[0], loop])
    dst = jnp.concatenate([edge_index[1], loop])
    etot = e + n
    grain = NC * NS * CHUNK
    ep = -(-etot // grain) * grain
    pad = ep - etot
    srcp = jnp.concatenate([src, jnp.full((pad,), n, src.dtype)])
    dstp = jnp.concatenate([dst, jnp.full((pad,), n, dst.dtype)])
    ch_per_tile = ep // grain
    srcp = srcp.reshape(NC * NS, ch_per_tile, CHUNK).astype(jnp.int32)
    dstp = dstp.reshape(NC * NS, ch_per_tile, CHUNK).astype(jnp.int32)

    x_pad = jnp.zeros((np_rows, din), x.dtype).at[:n].set(x)
    batch_p = jnp.full((1, np_rows), jnp.int32(1 << 20)).at[0, :n].set(
        batch_index.astype(jnp.int32))

    nh, zdim, hdim = heads_W1.shape
    w1cat = jnp.transpose(heads_W1, (1, 0, 2)).reshape(zdim, nh * hdim)
    b1cat = heads_b1.reshape(1, nh * hdim)
    w2blk = (heads_W2[:, :, 0][:, :, None]
             * jnp.eye(nh, dtype=heads_W2.dtype)[:, None, :]).reshape(
                 nh * hdim, nh)
    b2row = heads_b2[:, 0].reshape(1, nh)

    ones128 = jnp.ones((128, 128), jnp.float32)
    degt = _make_deg_kernel(np_rows, ch_per_tile, rpt, 128)(dstp, ones128)

    xs = pl.pallas_call(
        _tc_prescale,
        out_shape=jax.ShapeDtypeStruct((np_rows, d1), jnp.float32),
    )(x_pad, W1, degt)

    agg1 = _make_agg_kernel(np_rows, ch_per_tile, rpt, d1)(srcp, dstp, xs)

    ys = pl.pallas_call(
        _tc_layer2,
        out_shape=jax.ShapeDtypeStruct((np_rows, d1), jnp.float32),
    )(agg1, degt, b1.reshape(1, d1))

    agg2 = _make_agg_kernel(np_rows, ch_per_tile, rpt, d1)(srcp, dstp, ys)

    out = pl.pallas_call(
        _tc_head,
        out_shape=jax.ShapeDtypeStruct((g, nh), jnp.float32),
    )(agg2, degt, batch_p, solvent_descriptors, W2, b2.reshape(1, d2),
      lin1_W[:d2], lin1_W[d2:], lin1_b.reshape(1, -1),
      lin2_W, lin2_b.reshape(1, -1), lin3_W, lin3_b.reshape(1, -1),
      w1cat, b1cat, w2blk, b2row)
    return out


# speedup vs baseline: 24.6693x; 1.5282x over previous
import functools

import jax
import jax.numpy as jnp
from jax import lax
from jax.experimental import pallas as pl
from jax.experimental.pallas import tpu as pltpu
from jax.experimental.pallas import tpu_sc as plsc

NC = 2
NS = 16
LANES = 16
CHUNK = 128


def _sc_mesh():
    return plsc.VectorSubcoreMesh(core_axis_name="c", subcore_axis_name="s")


def _zero_fill(ref, rows, width):
    def body(i, carry):
        for k in range(width // LANES):
            ref[i, pl.ds(k * LANES, LANES)] = jnp.zeros((LANES,), jnp.float32)
        return carry
    lax.fori_loop(0, rows, body, 0)


def _zero_shared_rows(fill_v, acc_s, base, rpt):
    nfull, rem = rpt // 128, rpt % 128
    for k in range(nfull):
        pltpu.sync_copy(fill_v, acc_s.at[pl.ds(base + 128 * k, 128)])
    if rem:
        pltpu.sync_copy(fill_v.at[pl.ds(0, rem)],
                        acc_s.at[pl.ds(base + 128 * nfull, rem)])


def _make_deg_kernel(np_rows, ch_per_tile, rpt, width=16, tc_tiling=None):
    @functools.partial(
        pl.kernel,
        out_type=jax.ShapeDtypeStruct((NC, np_rows, width), jnp.float32),
        mesh=_sc_mesh(),
        compiler_params=pltpu.CompilerParams(use_tc_tiling_on_sc=tc_tiling),
        scratch_types=[
            pltpu.VMEM((ch_per_tile, CHUNK), jnp.int32),
            pltpu.VMEM((128, width), jnp.float32),
            pltpu.VMEM((128, width), jnp.float32),
            pltpu.VMEM_SHARED((np_rows, width), jnp.float32),
        ],
    )
    def deg_kernel(dst_hbm, ones_hbm, out_hbm, idx_v, fill_v, ones_v, acc_s):
        c = lax.axis_index("c")
        s = lax.axis_index("s")
        wid = s * NC + c
        base = s * rpt
        _zero_fill(fill_v, 128, width)
        _zero_shared_rows(fill_v, acc_s, base, rpt)
        pltpu.sync_copy(ones_hbm, ones_v)
        plsc.subcore_barrier()
        pltpu.sync_copy(dst_hbm.at[wid], idx_v)

        def step(j, carry):
            pltpu.sync_copy(ones_v, acc_s.at[idx_v.at[j]], add=True)
            return carry
        lax.fori_loop(0, ch_per_tile, step, 0)
        plsc.subcore_barrier()
        pltpu.sync_copy(acc_s.at[pl.ds(base, rpt)],
                        out_hbm.at[c].at[pl.ds(base, rpt)])

    return deg_kernel


def _make_agg_kernel(np_rows, ch_per_tile, rpt, width, tc_tiling=None,
                     split=False):
    nch = ch_per_tile * (NC if split else 1)

    @functools.partial(
        pl.kernel,
        out_type=jax.ShapeDtypeStruct((NC, np_rows, width), jnp.float32),
        mesh=_sc_mesh(),
        compiler_params=pltpu.CompilerParams(use_tc_tiling_on_sc=tc_tiling),
        scratch_types=[
            pltpu.VMEM((nch, CHUNK), jnp.int32),
            pltpu.VMEM((nch, CHUNK), jnp.int32),
            pltpu.VMEM((128, width), jnp.float32),
            pltpu.VMEM((128, width), jnp.float32),
            pltpu.VMEM_SHARED((np_rows, width), jnp.float32),
            pltpu.SemaphoreType.DMA,
            pltpu.SemaphoreType.DMA,
        ],
    )
    def agg_kernel(src_hbm, dst_hbm, tab_hbm, out_hbm,
                   idxs_v, idxd_v, rows_a, rows_b, acc_s, sem_a, sem_b):
        c = lax.axis_index("c")
        s = lax.axis_index("s")
        wid = s if split else s * NC + c
        tab = tab_hbm.at[c] if split else tab_hbm
        base = s * rpt
        _zero_fill(rows_a, 128, width)
        _zero_shared_rows(rows_a, acc_s, base, rpt)
        plsc.subcore_barrier()
        pltpu.sync_copy(src_hbm.at[wid], idxs_v)
        pltpu.sync_copy(dst_hbm.at[wid], idxd_v)

        def gather(j, buf, sem):
            return pltpu.async_copy(tab.at[idxs_v.at[j]], buf, sem)

        def gwait(j, buf, sem):
            pltpu.make_async_copy(tab.at[idxs_v.at[j]], buf, sem).wait()

        def scat(j, buf):
            pltpu.sync_copy(buf, acc_s.at[idxd_v.at[j]], add=True)

        gather(0, rows_a, sem_a)

        def pair(jj, carry):
            j = 2 * jj
            gather(j + 1, rows_b, sem_b)
            gwait(j, rows_a, sem_a)
            scat(j, rows_a)

            @pl.when(j + 2 < nch)
            def _():
                gather(j + 2, rows_a, sem_a)
            gwait(j + 1, rows_b, sem_b)
            scat(j + 1, rows_b)
            return carry
        lax.fori_loop(0, nch // 2, pair, 0)
        if nch % 2 == 1:
            gwait(nch - 1, rows_a, sem_a)
            scat(nch - 1, rows_a)
        plsc.subcore_barrier()
        pltpu.sync_copy(acc_s.at[pl.ds(base, rpt)],
                        out_hbm.at[c].at[pl.ds(base, rpt)])

    return agg_kernel


def _dinv_from_deg(degt_ref):
    deg = degt_ref[0, :, 0:1] + degt_ref[1, :, 0:1]
    return lax.rsqrt(jnp.maximum(deg, 1e-12))


def _tc_prescale(x_ref, w1_ref, degt_ref, xs_ref):
    dinv = _dinv_from_deg(degt_ref)
    xw = jnp.dot(x_ref[...], w1_ref[...], preferred_element_type=jnp.float32)
    xw = xw * dinv
    h = xs_ref.shape[2]
    xs_ref[0] = xw[:, :h]
    xs_ref[1] = xw[:, h:]


def _tc_layer2(agg_ref, degt_ref, w2_ref, b1_ref, ys_ref):
    dinv = _dinv_from_deg(degt_ref)
    h = agg_ref.shape[2]
    h1a = jnp.maximum(dinv * agg_ref[0] + b1_ref[:, :h], 0.0)
    h1b = jnp.maximum(dinv * agg_ref[1] + b1_ref[:, h:], 0.0)
    ys_ref[...] = (jnp.dot(h1a, w2_ref[:h], preferred_element_type=jnp.float32)
                   + jnp.dot(h1b, w2_ref[h:],
                             preferred_element_type=jnp.float32)) * dinv


def _tc_head(agg2_ref, degt_ref, batch_ref, solv_ref, b2_ref,
             l1a_ref, l1b_ref, l1bias_ref, l2w_ref, l2bias_ref,
             l3w_ref, l3bias_ref, w1cat_ref, b1cat_ref, w2blk_ref, b2row_ref,
             out_ref):
    num_graphs = out_ref.shape[0]
    np_rows = batch_ref.shape[1]
    dinv = _dinv_from_deg(degt_ref)
    nodes = dinv * (agg2_ref[0] + agg2_ref[1])
    gids = lax.broadcasted_iota(jnp.int32, (num_graphs, np_rows), 0)
    mask = (batch_ref[...] == gids).astype(jnp.float32)
    pooled = jnp.dot(mask, nodes, preferred_element_type=jnp.float32)
    cnt = jnp.sum(mask, axis=1, keepdims=True)
    pooled = pooled + cnt * b2_ref[...]
    z = jnp.dot(pooled, l1a_ref[...], preferred_element_type=jnp.float32)
    z = z + jnp.dot(solv_ref[...], l1b_ref[...],
                    preferred_element_type=jnp.float32)
    z = jnp.maximum(z + l1bias_ref[...], 0.0)
    z = jnp.maximum(jnp.dot(z, l2w_ref[...],
                            preferred_element_type=jnp.float32)
                    + l2bias_ref[...], 0.0)
    z = jnp.maximum(jnp.dot(z, l3w_ref[...],
                            preferred_element_type=jnp.float32)
                    + l3bias_ref[...], 0.0)
    hh = jnp.maximum(jnp.dot(z, w1cat_ref[...],
                             preferred_element_type=jnp.float32)
                     + b1cat_ref[...], 0.0)
    out_ref[...] = jnp.dot(hh, w2blk_ref[...],
                           preferred_element_type=jnp.float32) + b2row_ref[...]


def kernel(x, edge_index, edge_attr, batch_index, solvent_descriptors,
           mol_fingerprints, W1, b1, W2, b2, lin1_W, lin1_b, lin2_W, lin2_b,
           lin3_W, lin3_b, heads_W1, heads_b1, heads_W2, heads_b2):
    n, din = x.shape
    e = edge_index.shape[1]
    g = solvent_descriptors.shape[0]
    d1 = W1.shape[1]
    d2 = W2.shape[1]

    rpt = -(-(n + 1) // (NS * 8)) * 8
    np_rows = rpt * NS

    loop = jnp.arange(n, dtype=edge_index.dtype)
    src = jnp.concatenate([edge_index[0], loop])
    dst = jnp.concatenate([edge_index[1], loop])
    etot = e + n
    grain = NC * NS * CHUNK
    ep = -(-etot // grain) * grain
    pad = ep - etot
    srcp = jnp.concatenate([src, jnp.full((pad,), n, src.dtype)])
    dstp = jnp.concatenate([dst, jnp.full((pad,), n, dst.dtype)])
    ch_per_tile = ep // grain
    srcp = srcp.reshape(NC * NS, ch_per_tile, CHUNK).astype(jnp.int32)
    dstp = dstp.reshape(NC * NS, ch_per_tile, CHUNK).astype(jnp.int32)

    x_pad = jnp.zeros((np_rows, din), x.dtype).at[:n].set(x)
    batch_p = jnp.full((1, np_rows), jnp.int32(1 << 20)).at[0, :n].set(
        batch_index.astype(jnp.int32))

    nh, zdim, hdim = heads_W1.shape
    w1cat = jnp.transpose(heads_W1, (1, 0, 2)).reshape(zdim, nh * hdim)
    b1cat = heads_b1.reshape(1, nh * hdim)
    w2blk = (heads_W2[:, :, 0][:, :, None]
             * jnp.eye(nh, dtype=heads_W2.dtype)[:, None, :]).reshape(
                 nh * hdim, nh)
    b2row = heads_b2[:, 0].reshape(1, nh)

    ones16 = jnp.ones((128, 16), jnp.float32)
    degt = _make_deg_kernel(np_rows, ch_per_tile, rpt, 16,
                            tc_tiling=False)(dstp, ones16)

    xs = pl.pallas_call(
        _tc_prescale,
        out_shape=jax.ShapeDtypeStruct((NC, np_rows, d1 // NC), jnp.float32),
    )(x_pad, W1, degt)

    srcp_s = srcp.reshape(NS, NC * ch_per_tile, CHUNK)
    dstp_s = dstp.reshape(NS, NC * ch_per_tile, CHUNK)
    agg1 = _make_agg_kernel(np_rows, ch_per_tile, rpt, d1 // NC,
                            tc_tiling=False, split=True)(srcp_s, dstp_s, xs)

    ys = pl.pallas_call(
        _tc_layer2,
        out_shape=jax.ShapeDtypeStruct((np_rows, d2), jnp.float32),
    )(agg1, degt, W2, b1.reshape(1, d1))

    agg2 = _make_agg_kernel(np_rows, ch_per_tile, rpt, d2,
                            tc_tiling=False)(srcp, dstp, ys)

    out = pl.pallas_call(
        _tc_head,
        out_shape=jax.ShapeDtypeStruct((g, nh), jnp.float32),
    )(agg2, degt, batch_p, solvent_descriptors, b2.reshape(1, d2),
      lin1_W[:d2], lin1_W[d2:], lin1_b.reshape(1, -1),
      lin2_W, lin2_b.reshape(1, -1), lin3_W, lin3_b.reshape(1, -1),
      w1cat, b1cat, w2blk, b2row)
    return out


# speedup vs baseline: 25.5785x; 1.0369x over previous
import functools

import jax
import jax.numpy as jnp
from jax import lax
from jax.experimental import pallas as pl
from jax.experimental.pallas import tpu as pltpu
from jax.experimental.pallas import tpu_sc as plsc

NC = 2
NS = 16
LANES = 16
CHUNK = 128


def _sc_mesh():
    return plsc.VectorSubcoreMesh(core_axis_name="c", subcore_axis_name="s")


def _zero_fill(ref, rows, width):
    def body(i, carry):
        for k in range(width // LANES):
            ref[i, pl.ds(k * LANES, LANES)] = jnp.zeros((LANES,), jnp.float32)
        return carry
    lax.fori_loop(0, rows, body, 0)


def _zero_shared_rows(fill_v, acc_s, base, rpt):
    nfull, rem = rpt // 128, rpt % 128
    for k in range(nfull):
        pltpu.sync_copy(fill_v, acc_s.at[pl.ds(base + 128 * k, 128)])
    if rem:
        pltpu.sync_copy(fill_v.at[pl.ds(0, rem)],
                        acc_s.at[pl.ds(base + 128 * nfull, rem)])


def _make_deg_kernel(np_rows, ch_per_tile, rpt, width=16, tc_tiling=None):
    @functools.partial(
        pl.kernel,
        out_type=jax.ShapeDtypeStruct((NC, np_rows, width), jnp.float32),
        mesh=_sc_mesh(),
        compiler_params=pltpu.CompilerParams(use_tc_tiling_on_sc=tc_tiling),
        scratch_types=[
            pltpu.VMEM((ch_per_tile, CHUNK), jnp.int32),
            pltpu.VMEM((128, width), jnp.float32),
            pltpu.VMEM((128, width), jnp.float32),
            pltpu.VMEM_SHARED((np_rows, width), jnp.float32),
        ],
    )
    def deg_kernel(dst_hbm, ones_hbm, out_hbm, idx_v, fill_v, ones_v, acc_s):
        c = lax.axis_index("c")
        s = lax.axis_index("s")
        wid = s * NC + c
        base = s * rpt
        _zero_fill(fill_v, 128, width)
        _zero_shared_rows(fill_v, acc_s, base, rpt)
        pltpu.sync_copy(ones_hbm, ones_v)
        plsc.subcore_barrier()
        pltpu.sync_copy(dst_hbm.at[wid], idx_v)

        def step(j, carry):
            pltpu.sync_copy(ones_v, acc_s.at[idx_v.at[j]], add=True)
            return carry
        lax.fori_loop(0, ch_per_tile, step, 0)
        plsc.subcore_barrier()
        pltpu.sync_copy(acc_s.at[pl.ds(base, rpt)],
                        out_hbm.at[c].at[pl.ds(base, rpt)])

    return deg_kernel


def _make_agg_kernel(np_rows, ch_per_tile, rpt, width, tc_tiling=None,
                     split=False):
    nch = ch_per_tile * (NC if split else 1)

    @functools.partial(
        pl.kernel,
        out_type=jax.ShapeDtypeStruct((NC, np_rows, width), jnp.float32),
        mesh=_sc_mesh(),
        compiler_params=pltpu.CompilerParams(use_tc_tiling_on_sc=tc_tiling),
        scratch_types=[
            pltpu.VMEM((nch, CHUNK), jnp.int32),
            pltpu.VMEM((nch, CHUNK), jnp.int32),
        ] + [pltpu.VMEM((128, width), jnp.float32)] * 4
          + [pltpu.VMEM_SHARED((np_rows, width), jnp.float32)]
          + [pltpu.SemaphoreType.DMA] * 8,
    )
    def agg_kernel(src_hbm, dst_hbm, tab_hbm, out_hbm,
                   idxs_v, idxd_v, r0, r1, r2, r3, acc_s,
                   g0, g1, g2, g3, s0, s1, s2, s3):
        c = lax.axis_index("c")
        s = lax.axis_index("s")
        wid = s if split else s * NC + c
        tab = tab_hbm.at[c] if split else tab_hbm
        base = s * rpt
        bufs = (r0, r1, r2, r3)
        gsems = (g0, g1, g2, g3)
        ssems = (s0, s1, s2, s3)
        _zero_fill(r0, 128, width)
        _zero_shared_rows(r0, acc_s, base, rpt)
        plsc.subcore_barrier()
        pltpu.sync_copy(src_hbm.at[wid], idxs_v)
        pltpu.sync_copy(dst_hbm.at[wid], idxd_v)

        def gather(j, o):
            pltpu.async_copy(tab.at[idxs_v.at[j]], bufs[o], gsems[o])

        def gwait(j, o):
            pltpu.make_async_copy(tab.at[idxs_v.at[j]], bufs[o],
                                  gsems[o]).wait()

        def scat(j, o):
            pltpu.async_copy(bufs[o], acc_s.at[idxd_v.at[j]], ssems[o],
                             add=True)

        def swait(j, o):
            pltpu.make_async_copy(bufs[o], acc_s.at[idxd_v.at[j]],
                                  ssems[o]).wait()

        for o in range(min(4, nch)):
            gather(o, o)
        ngrp = max((nch - 4) // 4, 0)

        def group(k, carry):
            j = 4 * k
            for o in range(4):
                gwait(j + o, o)
                scat(j + o, o)
            for o in range(4):
                swait(j + o, o)
                gather(j + o + 4, o)
            return carry
        lax.fori_loop(0, ngrp, group, 0)
        for r in range(4 * ngrp, nch):
            o = r % 4
            if r >= 4 * ngrp + 4:
                swait(r - 4, o)
                gather(r, o)
            gwait(r, o)
            scat(r, o)
        for o in range(4):
            lasts = [r for r in range(4 * ngrp, nch) if r % 4 == o]
            if lasts:
                swait(lasts[-1], o)
        plsc.subcore_barrier()
        pltpu.sync_copy(acc_s.at[pl.ds(base, rpt)],
                        out_hbm.at[c].at[pl.ds(base, rpt)])

    return agg_kernel


def _dinv_from_deg(degt_ref):
    deg = degt_ref[0, :, 0:1] + degt_ref[1, :, 0:1]
    return lax.rsqrt(jnp.maximum(deg, 1e-12))


def _tc_prescale(x_ref, w1_ref, degt_ref, xs_ref):
    dinv = _dinv_from_deg(degt_ref)
    xw = jnp.dot(x_ref[...], w1_ref[...], preferred_element_type=jnp.float32)
    xw = xw * dinv
    h = xs_ref.shape[2]
    xs_ref[0] = xw[:, :h]
    xs_ref[1] = xw[:, h:]


def _tc_layer2(agg_ref, degt_ref, w2_ref, b1_ref, ys_ref):
    dinv = _dinv_from_deg(degt_ref)
    h = agg_ref.shape[2]
    h1a = jnp.maximum(dinv * agg_ref[0] + b1_ref[:, :h], 0.0)
    h1b = jnp.maximum(dinv * agg_ref[1] + b1_ref[:, h:], 0.0)
    ys_ref[...] = (jnp.dot(h1a, w2_ref[:h], preferred_element_type=jnp.float32)
                   + jnp.dot(h1b, w2_ref[h:],
                             preferred_element_type=jnp.float32)) * dinv


def _tc_head(agg2_ref, degt_ref, batch_ref, solv_ref, b2_ref,
             l1a_ref, l1b_ref, l1bias_ref, l2w_ref, l2bias_ref,
             l3w_ref, l3bias_ref, w1cat_ref, b1cat_ref, w2blk_ref, b2row_ref,
             out_ref):
    num_graphs = out_ref.shape[0]
    np_rows = batch_ref.shape[1]
    dinv = _dinv_from_deg(degt_ref)
    nodes = dinv * (agg2_ref[0] + agg2_ref[1])
    gids = lax.broadcasted_iota(jnp.int32, (num_graphs, np_rows), 0)
    mask = (batch_ref[...] == gids).astype(jnp.float32)
    pooled = jnp.dot(mask, nodes, preferred_element_type=jnp.float32)
    cnt = jnp.sum(mask, axis=1, keepdims=True)
    pooled = pooled + cnt * b2_ref[...]
    z = jnp.dot(pooled, l1a_ref[...], preferred_element_type=jnp.float32)
    z = z + jnp.dot(solv_ref[...], l1b_ref[...],
                    preferred_element_type=jnp.float32)
    z = jnp.maximum(z + l1bias_ref[...], 0.0)
    z = jnp.maximum(jnp.dot(z, l2w_ref[...],
                            preferred_element_type=jnp.float32)
                    + l2bias_ref[...], 0.0)
    z = jnp.maximum(jnp.dot(z, l3w_ref[...],
                            preferred_element_type=jnp.float32)
                    + l3bias_ref[...], 0.0)
    hh = jnp.maximum(jnp.dot(z, w1cat_ref[...],
                             preferred_element_type=jnp.float32)
                     + b1cat_ref[...], 0.0)
    out_ref[...] = jnp.dot(hh, w2blk_ref[...],
                           preferred_element_type=jnp.float32) + b2row_ref[...]


def kernel(x, edge_index, edge_attr, batch_index, solvent_descriptors,
           mol_fingerprints, W1, b1, W2, b2, lin1_W, lin1_b, lin2_W, lin2_b,
           lin3_W, lin3_b, heads_W1, heads_b1, heads_W2, heads_b2):
    n, din = x.shape
    e = edge_index.shape[1]
    g = solvent_descriptors.shape[0]
    d1 = W1.shape[1]
    d2 = W2.shape[1]

    rpt = -(-(n + 1) // (NS * 8)) * 8
    np_rows = rpt * NS

    loop = jnp.arange(n, dtype=edge_index.dtype)
    src = jnp.concatenate([edge_index[0], loop])
    dst = jnp.concatenate([edge_index[1], loop])
    etot = e + n
    grain = NC * NS * CHUNK
    ep = -(-etot // grain) * grain
    pad = ep - etot
    srcp = jnp.concatenate([src, jnp.full((pad,), n, src.dtype)])
    dstp = jnp.concatenate([dst, jnp.full((pad,), n, dst.dtype)])
    ch_per_tile = ep // grain
    srcp = srcp.reshape(NC * NS, ch_per_tile, CHUNK).astype(jnp.int32)
    dstp = dstp.reshape(NC * NS, ch_per_tile, CHUNK).astype(jnp.int32)

    x_pad = jnp.zeros((np_rows, din), x.dtype).at[:n].set(x)
    batch_p = jnp.full((1, np_rows), jnp.int32(1 << 20)).at[0, :n].set(
        batch_index.astype(jnp.int32))

    nh, zdim, hdim = heads_W1.shape
    w1cat = jnp.transpose(heads_W1, (1, 0, 2)).reshape(zdim, nh * hdim)
    b1cat = heads_b1.reshape(1, nh * hdim)
    w2blk = (heads_W2[:, :, 0][:, :, None]
             * jnp.eye(nh, dtype=heads_W2.dtype)[:, None, :]).reshape(
                 nh * hdim, nh)
    b2row = heads_b2[:, 0].reshape(1, nh)

    ones16 = jnp.ones((128, 16), jnp.float32)
    degt = _make_deg_kernel(np_rows, ch_per_tile, rpt, 16,
                            tc_tiling=False)(dstp, ones16)

    xs = pl.pallas_call(
        _tc_prescale,
        out_shape=jax.ShapeDtypeStruct((NC, np_rows, d1 // NC), jnp.float32),
    )(x_pad, W1, degt)

    srcp_s = srcp.reshape(NS, NC * ch_per_tile, CHUNK)
    dstp_s = dstp.reshape(NS, NC * ch_per_tile, CHUNK)
    agg1 = _make_agg_kernel(np_rows, ch_per_tile, rpt, d1 // NC,
                            tc_tiling=False, split=True)(srcp_s, dstp_s, xs)

    ys = pl.pallas_call(
        _tc_layer2,
        out_shape=jax.ShapeDtypeStruct((np_rows, d2), jnp.float32),
    )(agg1, degt, W2, b1.reshape(1, d1))

    agg2 = _make_agg_kernel(np_rows, ch_per_tile, rpt, d2,
                            tc_tiling=False)(srcp, dstp, ys)

    out = pl.pallas_call(
        _tc_head,
        out_shape=jax.ShapeDtypeStruct((g, nh), jnp.float32),
    )(agg2, degt, batch_p, solvent_descriptors, b2.reshape(1, d2),
      lin1_W[:d2], lin1_W[d2:], lin1_b.reshape(1, -1),
      lin2_W, lin2_b.reshape(1, -1), lin3_W, lin3_b.reshape(1, -1),
      w1cat, b1cat, w2blk, b2row)
    return out


# speedup vs baseline: 26.8850x; 1.0511x over previous
import functools

import jax
import jax.numpy as jnp
from jax import lax
from jax.experimental import pallas as pl
from jax.experimental.pallas import tpu as pltpu
from jax.experimental.pallas import tpu_sc as plsc

NC = 2
NS = 16
LANES = 16
CHUNK = 128


def _sc_mesh():
    return plsc.VectorSubcoreMesh(core_axis_name="c", subcore_axis_name="s")


def _zero_fill(ref, rows, width):
    def body(i, carry):
        for k in range(width // LANES):
            ref[i, pl.ds(k * LANES, LANES)] = jnp.zeros((LANES,), jnp.float32)
        return carry
    lax.fori_loop(0, rows, body, 0)


def _zero_shared_rows(fill_v, nfill, acc_s, base, rpt):
    nfull, rem = rpt // nfill, rpt % nfill
    for k in range(nfull):
        pltpu.sync_copy(fill_v, acc_s.at[pl.ds(base + nfill * k, nfill)])
    if rem:
        pltpu.sync_copy(fill_v.at[pl.ds(0, rem)],
                        acc_s.at[pl.ds(base + nfill * nfull, rem)])


def _make_deg_kernel(np_rows, ch_per_tile, rpt, width=16, tc_tiling=None):
    @functools.partial(
        pl.kernel,
        out_type=jax.ShapeDtypeStruct((NC, np_rows, width), jnp.float32),
        mesh=_sc_mesh(),
        compiler_params=pltpu.CompilerParams(use_tc_tiling_on_sc=tc_tiling),
        scratch_types=[
            pltpu.VMEM((ch_per_tile, CHUNK), jnp.int32),
            pltpu.VMEM((128, width), jnp.float32),
            pltpu.VMEM((128, width), jnp.float32),
            pltpu.VMEM_SHARED((np_rows, width), jnp.float32),
        ],
    )
    def deg_kernel(dst_hbm, ones_hbm, out_hbm, idx_v, fill_v, ones_v, acc_s):
        c = lax.axis_index("c")
        s = lax.axis_index("s")
        wid = s * NC + c
        base = s * rpt
        _zero_fill(fill_v, 128, width)
        _zero_shared_rows(fill_v, 128, acc_s, base, rpt)
        pltpu.sync_copy(ones_hbm, ones_v)
        plsc.subcore_barrier()
        pltpu.sync_copy(dst_hbm.at[wid], idx_v)

        def step(j, carry):
            pltpu.sync_copy(ones_v, acc_s.at[idx_v.at[j]], add=True)
            return carry
        lax.fori_loop(0, ch_per_tile, step, 0)
        plsc.subcore_barrier()
        pltpu.sync_copy(acc_s.at[pl.ds(base, rpt)],
                        out_hbm.at[c].at[pl.ds(base, rpt)])

    return deg_kernel


def _make_agg_kernel(np_rows, ch_per_tile, rpt, width, tc_tiling=None,
                     split=False):
    nch = ch_per_tile * (NC if split else 1)
    grp = 2
    assert nch % (2 * grp) == grp, "chunk count sized so the tail is one group"
    assert nch >= 3 * grp

    @functools.partial(
        pl.kernel,
        out_type=jax.ShapeDtypeStruct((NC, np_rows, width), jnp.float32),
        mesh=_sc_mesh(),
        compiler_params=pltpu.CompilerParams(use_tc_tiling_on_sc=tc_tiling),
        scratch_types=[
            pltpu.VMEM((nch, CHUNK), jnp.int32),
            pltpu.VMEM((nch, CHUNK), jnp.int32),
            pltpu.VMEM((grp * 128, width), jnp.float32),
            pltpu.VMEM((grp * 128, width), jnp.float32),
            pltpu.VMEM_SHARED((np_rows, width), jnp.float32),
        ] + [pltpu.SemaphoreType.DMA] * 4,
    )
    def agg_kernel(src_hbm, dst_hbm, tab_hbm, out_hbm,
                   idxs_v, idxd_v, big_a, big_b, acc_s, ga, gb, sa, sb):
        c = lax.axis_index("c")
        s = lax.axis_index("s")
        wid = s if split else s * NC + c
        tab = tab_hbm.at[c] if split else tab_hbm
        base = s * rpt
        _zero_fill(big_a, grp * 128, width)
        _zero_shared_rows(big_a, grp * 128, acc_s, base, rpt)
        plsc.subcore_barrier()
        pltpu.sync_copy(src_hbm.at[wid], idxs_v)
        pltpu.sync_copy(dst_hbm.at[wid], idxd_v)

        def ggrp(j, big, sem):
            for o in range(grp):
                pltpu.async_copy(tab.at[idxs_v.at[j + o]],
                                 big.at[pl.ds(o * 128, 128)], sem)

        def gwaitgrp(big, sem):
            pltpu.make_async_copy(tab.at[pl.ds(0, grp * 128)], big,
                                  sem).wait()

        def scatgrp(j, big, sem):
            for o in range(grp):
                pltpu.async_copy(big.at[pl.ds(o * 128, 128)],
                                 acc_s.at[idxd_v.at[j + o]], sem, add=True)

        def swaitgrp(big, sem):
            pltpu.make_async_copy(big, acc_s.at[pl.ds(0, grp * 128)],
                                  sem).wait()

        ggrp(0, big_a, ga)

        def body(k, carry):
            j = 2 * grp * k
            ggrp(j + grp, big_b, gb)
            gwaitgrp(big_a, ga)
            scatgrp(j, big_a, sa)
            swaitgrp(big_a, sa)
            ggrp(j + 2 * grp, big_a, ga)
            gwaitgrp(big_b, gb)
            scatgrp(j + grp, big_b, sb)
            swaitgrp(big_b, sb)
            return carry
        lax.fori_loop(0, (nch - grp) // (2 * grp), body, 0)
        gwaitgrp(big_a, ga)
        scatgrp(nch - grp, big_a, sa)
        swaitgrp(big_a, sa)
        plsc.subcore_barrier()
        pltpu.sync_copy(acc_s.at[pl.ds(base, rpt)],
                        out_hbm.at[c].at[pl.ds(base, rpt)])

    return agg_kernel


def _dinv_from_deg(degt_ref):
    deg = degt_ref[0, :, 0:1] + degt_ref[1, :, 0:1]
    return lax.rsqrt(jnp.maximum(deg, 1e-12))


def _tc_prescale(x_ref, w1_ref, degt_ref, xs_ref):
    dinv = _dinv_from_deg(degt_ref)
    xw = jnp.dot(x_ref[...], w1_ref[...], preferred_element_type=jnp.float32)
    xw = xw * dinv
    h = xs_ref.shape[2]
    xs_ref[0] = xw[:, :h]
    xs_ref[1] = xw[:, h:]


def _tc_layer2(agg_ref, degt_ref, w2_ref, b1_ref, ys_ref):
    dinv = _dinv_from_deg(degt_ref)
    h = agg_ref.shape[2]
    h1a = jnp.maximum(dinv * agg_ref[0] + b1_ref[:, :h], 0.0)
    h1b = jnp.maximum(dinv * agg_ref[1] + b1_ref[:, h:], 0.0)
    ys = (jnp.dot(h1a, w2_ref[:h], preferred_element_type=jnp.float32)
          + jnp.dot(h1b, w2_ref[h:],
                    preferred_element_type=jnp.float32)) * dinv
    h2 = ys_ref.shape[2]
    ys_ref[0] = ys[:, :h2]
    ys_ref[1] = ys[:, h2:]


def _tc_head(agg2_ref, degt_ref, batch_ref, solv_ref, b2_ref,
             l1a_ref, l1b_ref, l1bias_ref, l2w_ref, l2bias_ref,
             l3w_ref, l3bias_ref, w1cat_ref, b1cat_ref, w2blk_ref, b2row_ref,
             out_ref):
    num_graphs = out_ref.shape[0]
    np_rows = batch_ref.shape[1]
    dinv = _dinv_from_deg(degt_ref)
    h2 = agg2_ref.shape[2]
    nodes_a = dinv * agg2_ref[0]
    nodes_b = dinv * agg2_ref[1]
    gids = lax.broadcasted_iota(jnp.int32, (num_graphs, np_rows), 0)
    mask = (batch_ref[...] == gids).astype(jnp.float32)
    cnt = jnp.sum(mask, axis=1, keepdims=True)
    pooled_a = (jnp.dot(mask, nodes_a, preferred_element_type=jnp.float32)
                + cnt * b2_ref[:, :h2])
    pooled_b = (jnp.dot(mask, nodes_b, preferred_element_type=jnp.float32)
                + cnt * b2_ref[:, h2:])
    z = (jnp.dot(pooled_a, l1a_ref[:h2], preferred_element_type=jnp.float32)
         + jnp.dot(pooled_b, l1a_ref[h2:],
                   preferred_element_type=jnp.float32))
    z = z + jnp.dot(solv_ref[...], l1b_ref[...],
                    preferred_element_type=jnp.float32)
    z = jnp.maximum(z + l1bias_ref[...], 0.0)
    z = jnp.maximum(jnp.dot(z, l2w_ref[...],
                            preferred_element_type=jnp.float32)
                    + l2bias_ref[...], 0.0)
    z = jnp.maximum(jnp.dot(z, l3w_ref[...],
                            preferred_element_type=jnp.float32)
                    + l3bias_ref[...], 0.0)
    hh = jnp.maximum(jnp.dot(z, w1cat_ref[...],
                             preferred_element_type=jnp.float32)
                     + b1cat_ref[...], 0.0)
    out_ref[...] = jnp.dot(hh, w2blk_ref[...],
                           preferred_element_type=jnp.float32) + b2row_ref[...]


def kernel(x, edge_index, edge_attr, batch_index, solvent_descriptors,
           mol_fingerprints, W1, b1, W2, b2, lin1_W, lin1_b, lin2_W, lin2_b,
           lin3_W, lin3_b, heads_W1, heads_b1, heads_W2, heads_b2):
    n, din = x.shape
    e = edge_index.shape[1]
    g = solvent_descriptors.shape[0]
    d1 = W1.shape[1]
    d2 = W2.shape[1]

    rpt = -(-(n + 1) // (NS * 8)) * 8
    np_rows = rpt * NS

    loop = jnp.arange(n, dtype=edge_index.dtype)
    src = jnp.concatenate([edge_index[0], loop])
    dst = jnp.concatenate([edge_index[1], loop])
    etot = e + n
    grain = NC * NS * CHUNK
    nchk = -(-etot // grain)
    while nchk % 2 != 1:
        nchk += 1
    ep = nchk * grain
    pad = ep - etot
    srcp = jnp.concatenate([src, jnp.full((pad,), n, src.dtype)])
    dstp = jnp.concatenate([dst, jnp.full((pad,), n, dst.dtype)])
    ch_per_tile = ep // grain
    srcp = srcp.reshape(NC * NS, ch_per_tile, CHUNK).astype(jnp.int32)
    dstp = dstp.reshape(NC * NS, ch_per_tile, CHUNK).astype(jnp.int32)

    x_pad = jnp.zeros((np_rows, din), x.dtype).at[:n].set(x)
    batch_p = jnp.full((1, np_rows), jnp.int32(1 << 20)).at[0, :n].set(
        batch_index.astype(jnp.int32))

    nh, zdim, hdim = heads_W1.shape
    w1cat = jnp.transpose(heads_W1, (1, 0, 2)).reshape(zdim, nh * hdim)
    b1cat = heads_b1.reshape(1, nh * hdim)
    w2blk = (heads_W2[:, :, 0][:, :, None]
             * jnp.eye(nh, dtype=heads_W2.dtype)[:, None, :]).reshape(
                 nh * hdim, nh)
    b2row = heads_b2[:, 0].reshape(1, nh)

    ones16 = jnp.ones((128, 16), jnp.float32)
    degt = _make_deg_kernel(np_rows, ch_per_tile, rpt, 16,
                            tc_tiling=False)(dstp, ones16)

    xs = pl.pallas_call(
        _tc_prescale,
        out_shape=jax.ShapeDtypeStruct((NC, np_rows, d1 // NC), jnp.float32),
    )(x_pad, W1, degt)

    srcp_s = srcp.reshape(NS, NC * ch_per_tile, CHUNK)
    dstp_s = dstp.reshape(NS, NC * ch_per_tile, CHUNK)
    agg1 = _make_agg_kernel(np_rows, ch_per_tile, rpt, d1 // NC,
                            tc_tiling=False, split=True)(srcp_s, dstp_s, xs)

    ys = pl.pallas_call(
        _tc_layer2,
        out_shape=jax.ShapeDtypeStruct((NC, np_rows, d2 // NC), jnp.float32),
    )(agg1, degt, W2, b1.reshape(1, d1))

    agg2 = _make_agg_kernel(np_rows, ch_per_tile, rpt, d2 // NC,
                            tc_tiling=False, split=True)(srcp_s, dstp_s, ys)

    out = pl.pallas_call(
        _tc_head,
        out_shape=jax.ShapeDtypeStruct((g, nh), jnp.float32),
    )(agg2, degt, batch_p, solvent_descriptors, b2.reshape(1, d2),
      lin1_W[:d2], lin1_W[d2:], lin1_b.reshape(1, -1),
      lin2_W, lin2_b.reshape(1, -1), lin3_W, lin3_b.reshape(1, -1),
      w1cat, b1cat, w2blk, b2row)
    return out
